# baseline (speedup 1.0000x reference)
import functools

import jax
import jax.numpy as jnp
from jax import lax
from jax.experimental import pallas as pl
from jax.experimental.pallas import tpu as pltpu
from jax.experimental.pallas import tpu_sc as plsc

B = 16384
N_CAT = 26
N_DENSE = 13
NUM_BINS = 128
OUT_W = N_CAT * NUM_BINS + 3 * NUM_BINS + N_DENSE

NC, NS, L = 2, 16, 16
NW = NC * NS
RW = B // NW
R = 16
NCHUNK = RW // R

XI_LEN = RW * N_CAT
XF_LEN = RW * N_DENSE
BUF_LEN = R * OUT_W
BUF_PAD = BUF_LEN + 16


def _sc_body(xi_hbm, xf_hbm, out_hbm, xi_v, xf_v, buf_v, rec_v):
    wid = lax.axis_index("s") * NC + lax.axis_index("c")
    base = wid * RW

    pltpu.sync_copy(xi_hbm.at[pl.ds(base * N_CAT, XI_LEN)],
                    xi_v.at[pl.ds(0, XI_LEN)])
    pltpu.sync_copy(xf_hbm.at[pl.ds(base * N_DENSE, XF_LEN)],
                    xf_v.at[pl.ds(0, XF_LEN)])

    zeros = jnp.zeros((L,), jnp.float32)

    def _zero(i, _):
        buf_v[pl.ds(i * L, L)] = zeros
        return _

    lax.fori_loop(0, BUF_PAD // L, _zero, None)

    iot = lax.iota(jnp.int32, L)
    ones = jnp.ones((L,), jnp.float32)
    mask3 = iot < 3
    mask13 = iot < N_DENSE

    def _chunk(c, _):
        for r in range(R):
            row = c * R + r
            off = row * N_CAT
            vA = xi_v[pl.ds(off, L)]
            hA = (vA * 31 + 17) & 127
            idxA = (r * OUT_W) + iot * NUM_BINS + hA
            plsc.store_scatter(buf_v, [idxA], ones)
            vB = xi_v[pl.ds(off + (N_CAT - L), L)]
            hB = (vB * 31 + 17) & 127
            idxB = (r * OUT_W + (N_CAT - L) * NUM_BINS) + iot * NUM_BINS + hB
            plsc.store_scatter(buf_v, [idxB], ones)
            ge = off + iot * 2
            a = plsc.load_gather(xi_v, [ge]) % 32749
            b = plsc.load_gather(xi_v, [ge + 1]) % 32749
            comb = a * 32749 + b
            hc = (comb * 31 + 17) & 127
            idxC = (r * OUT_W + N_CAT * NUM_BINS) + iot * NUM_BINS + hc
            idxC = jnp.where(mask3, idxC, 0)
            plsc.store_scatter(buf_v, [idxC], ones, mask=mask3)
            fv = xf_v[pl.ds(row * N_DENSE, L)]
            idxD = (r * OUT_W + (N_CAT + 3) * NUM_BINS) + iot
            plsc.store_scatter(buf_v, [idxD], fv, mask=mask13)
            rec_v[pl.ds(r * 48, L)] = idxA
            rec_v[pl.ds(r * 48 + 16, L)] = idxB
            rec_v[pl.ds(r * 48 + 32, L)] = idxC

        pltpu.sync_copy(
            buf_v.at[pl.ds(0, BUF_LEN)],
            out_hbm.at[pl.ds((base + c * R) * OUT_W, BUF_LEN)],
        )

        for r in range(R):
            plsc.store_scatter(buf_v, [rec_v[pl.ds(r * 48, L)]], zeros)
            plsc.store_scatter(buf_v, [rec_v[pl.ds(r * 48 + 16, L)]], zeros)
            plsc.store_scatter(buf_v, [rec_v[pl.ds(r * 48 + 32, L)]], zeros,
                               mask=mask3)
        return _

    lax.fori_loop(0, NCHUNK, _chunk, None)


def kernel(int_features, float_features):
    xi = int_features.reshape(B * N_CAT)
    xf = float_features.reshape(B * N_DENSE)
    run = pl.kernel(
        _sc_body,
        out_type=jax.ShapeDtypeStruct((B * OUT_W,), jnp.float32),
        mesh=plsc.VectorSubcoreMesh(core_axis_name="c", subcore_axis_name="s"),
        compiler_params=pltpu.CompilerParams(needs_layout_passes=False),
        scratch_types=[
            pltpu.VMEM((XI_LEN + 32,), jnp.int32),
            pltpu.VMEM((XF_LEN + 16,), jnp.float32),
            pltpu.VMEM((BUF_PAD,), jnp.float32),
            pltpu.VMEM((R * 48,), jnp.int32),
        ],
    )
    return run(xi, xf).reshape(B, OUT_W)


# speedup vs baseline: 1.7079x; 1.7079x over previous
import jax
import jax.numpy as jnp
from jax import lax
from jax.experimental import pallas as pl
from jax.experimental.pallas import tpu as pltpu
from jax.experimental.pallas import tpu_sc as plsc

B = 16384
N_CAT = 26
N_DENSE = 13
NUM_BINS = 128
OUT_W = N_CAT * NUM_BINS + 3 * NUM_BINS + N_DENSE

NC, NS, L = 2, 16, 16
NW = NC * NS
RW = B // NW
R = 16
NCHUNK = RW // R

XI_LEN = RW * N_CAT
XF_LEN = RW * N_DENSE


def _sc_body(xi_hbm, xf_hbm, out_hbm, xi_v, xf_v, buf_v, rec_v):
    wid = lax.axis_index("s") * NC + lax.axis_index("c")
    base = wid * RW

    pltpu.sync_copy(xi_hbm.at[pl.ds(base * N_CAT, XI_LEN)],
                    xi_v.at[pl.ds(0, XI_LEN)])
    pltpu.sync_copy(xf_hbm.at[pl.ds(base * N_DENSE, XF_LEN)],
                    xf_v.at[pl.ds(0, XF_LEN)])

    zeros = jnp.zeros((L,), jnp.float32)
    iot = lax.iota(jnp.int32, L)
    ones = jnp.ones((L,), jnp.float32)
    mask3 = iot < 3
    mask13 = iot < N_DENSE

    def _zero(i, _):
        for r in range(R):
            buf_v[r, pl.ds(i * L, L)] = zeros
        return _

    lax.fori_loop(0, OUT_W // L, _zero, None)
    for r in range(R):
        tail = (OUT_W // L) * L
        tcol = jnp.where(iot < OUT_W - tail, tail + iot, 0)
        plsc.store_scatter(buf_v, [jnp.full((L,), r, jnp.int32), tcol],
                           zeros, mask=iot < OUT_W - tail)

    def _chunk(c, _):
        for r in range(R):
            rows = jnp.full((L,), r, jnp.int32)
            row = c * R + r
            off = row * N_CAT
            vA = xi_v[pl.ds(off, L)]
            hA = (vA * 31 + 17) & 127
            idxA = iot * NUM_BINS + hA
            plsc.store_scatter(buf_v, [rows, idxA], ones)
            vB = xi_v[pl.ds(off + (N_CAT - L), L)]
            hB = (vB * 31 + 17) & 127
            idxB = ((N_CAT - L) * NUM_BINS) + iot * NUM_BINS + hB
            plsc.store_scatter(buf_v, [rows, idxB], ones)
            ge = off + iot * 2
            a = plsc.load_gather(xi_v, [ge]) % 32749
            b = plsc.load_gather(xi_v, [ge + 1]) % 32749
            comb = a * 32749 + b
            hc = (comb * 31 + 17) & 127
            idxC = (N_CAT * NUM_BINS) + iot * NUM_BINS + hc
            idxC = jnp.where(mask3, idxC, 0)
            plsc.store_scatter(buf_v, [rows, idxC], ones, mask=mask3)
            fv = xf_v[pl.ds(row * N_DENSE, L)]
            idxD = jnp.where(mask13, ((N_CAT + 3) * NUM_BINS) + iot, 0)
            plsc.store_scatter(buf_v, [rows, idxD], fv, mask=mask13)
            rec_v[pl.ds(r * 48, L)] = idxA
            rec_v[pl.ds(r * 48 + 16, L)] = idxB
            rec_v[pl.ds(r * 48 + 32, L)] = idxC

        pltpu.sync_copy(buf_v, out_hbm.at[pl.ds(base + c * R, R)])

        for r in range(R):
            rows = jnp.full((L,), r, jnp.int32)
            plsc.store_scatter(buf_v, [rows, rec_v[pl.ds(r * 48, L)]], zeros)
            plsc.store_scatter(buf_v, [rows, rec_v[pl.ds(r * 48 + 16, L)]],
                               zeros)
            plsc.store_scatter(buf_v, [rows, rec_v[pl.ds(r * 48 + 32, L)]],
                               zeros, mask=mask3)
        return _

    lax.fori_loop(0, NCHUNK, _chunk, None)


def kernel(int_features, float_features):
    xi = int_features.reshape(B * N_CAT)
    xf = float_features.reshape(B * N_DENSE)
    run = pl.kernel(
        _sc_body,
        out_type=jax.ShapeDtypeStruct((B, OUT_W), jnp.float32),
        mesh=plsc.VectorSubcoreMesh(core_axis_name="c", subcore_axis_name="s"),
        compiler_params=pltpu.CompilerParams(needs_layout_passes=False),
        scratch_types=[
            pltpu.VMEM((XI_LEN + 32,), jnp.int32),
            pltpu.VMEM((XF_LEN + 16,), jnp.float32),
            pltpu.VMEM((R, OUT_W), jnp.float32),
            pltpu.VMEM((R * 48,), jnp.int32),
        ],
    )
    return run(xi, xf)


# speedup vs baseline: 6.2064x; 3.6340x over previous
import jax
import jax.numpy as jnp
from jax import lax
from jax.experimental import pallas as pl
from jax.experimental.pallas import tpu as pltpu
from jax.experimental.pallas import tpu_sc as plsc

B = 16384
N_CAT = 26
N_DENSE = 13
NUM_BINS = 128
N_ONEHOT = N_CAT + 3
OUT_W = N_ONEHOT * NUM_BINS + N_DENSE

NC, NS, L = 2, 16, 16
CB = 256
NCHUNK = B // CB
N_ITEMS = N_ONEHOT * NCHUNK
N_OH_WORKERS = 31
KMAX = 60
DENSE_ROW0 = N_ONEHOT * NUM_BINS


def _item_srcs(item):
    f = item >> 6
    b0 = (item & 63) * CB
    is_cross = f >= N_CAT
    p = f - N_CAT
    offa = jnp.where(is_cross, (2 * p) * B, f * B) + b0
    offb = jnp.where(is_cross, (2 * p + 1) * B, f * B) + b0
    return f, b0, offa, offb, is_cross


def _sc_body(xi_hbm, xf_hbm, out_hbm, stg0, stg1, buf0, buf1, hrec0, hrec1,
             dbuf, sin0, sin1, sout0, sout1):
    wid = lax.axis_index("s") * NC + lax.axis_index("c")
    iot = lax.iota(jnp.int32, L)
    ones = jnp.ones((L,), jnp.float32)
    zeros = jnp.zeros((L,), jnp.float32)
    izeros = jnp.zeros((L,), jnp.int32)

    stgs = (stg0, stg1)
    bufs = (buf0, buf1)
    hrecs = (hrec0, hrec1)
    sins = (sin0, sin1)
    souts = (sout0, sout1)

    @pl.when(wid == N_OH_WORKERS)
    def _dense():
        for t in range(N_DENSE):
            pltpu.sync_copy(xf_hbm.at[pl.ds(t, 1), pl.ds(0, B)], dbuf)
            pltpu.sync_copy(dbuf, out_hbm.at[pl.ds(DENSE_ROW0 + t, 1),
                                             pl.ds(0, B)])

    @pl.when(wid < N_OH_WORKERS)
    def _onehot():
        def _zero(i, _):
            for u in range(8):
                g = i * 8 + u
                buf0[g >> 4, pl.ds((g & 15) * L, L)] = zeros
                buf1[g >> 4, pl.ds((g & 15) * L, L)] = zeros
            return _

        lax.fori_loop(0, NUM_BINS * CB // L // 8, _zero, None)
        for g in range(CB // L):
            hrec0[pl.ds(g * L, L)] = izeros
            hrec1[pl.ds(g * L, L)] = izeros

        def _issue_in(item, s):
            _, _, offa, offb, _ = _item_srcs(item)
            pltpu.async_copy(xi_hbm.at[pl.ds(offa, CB)],
                             stgs[s].at[pl.ds(0, CB)], sins[s])
            pltpu.async_copy(xi_hbm.at[pl.ds(offb, CB)],
                             stgs[s].at[pl.ds(CB, CB)], sins[s])

        for s in range(2):
            item = wid + N_OH_WORKERS * s
            _issue_in(item, s)

        def _pair(kk, _):
            for s in range(2):
                k = 2 * kk + s
                item = wid + N_OH_WORKERS * k
                buf, stg, hrec = bufs[s], stgs[s], hrecs[s]

                @pl.when(item < N_ITEMS)
                def _run():
                    f, b0, _, _, is_cross = _item_srcs(item)
                    pltpu.make_async_copy(
                        xi_hbm.at[pl.ds(0, 2 * CB)], stg, sins[s]).wait()
                    @pl.when(k >= 2)
                    def _cleanup():
                        pltpu.make_async_copy(
                            buf, out_hbm.at[pl.ds(0, NUM_BINS), pl.ds(0, CB)],
                            souts[s]).wait()
                        for g in range(CB // L):
                            hold = hrec[pl.ds(g * L, L)]
                            plsc.store_scatter(buf, [hold, g * L + iot],
                                               zeros)

                    @pl.when(is_cross)
                    def _cross():
                        for g in range(CB // L):
                            xa = stg[pl.ds(g * L, L)]
                            xb = stg[pl.ds(CB + g * L, L)]
                            comb = (xa % 32749) * 32749 + (xb % 32749)
                            h = (comb * 31 + 17) & 127
                            plsc.store_scatter(buf, [h, g * L + iot], ones)
                            hrec[pl.ds(g * L, L)] = h

                    @pl.when(jnp.logical_not(is_cross))
                    def _plain():
                        for g in range(CB // L):
                            xa = stg[pl.ds(g * L, L)]
                            h = (xa * 31 + 17) & 127
                            plsc.store_scatter(buf, [h, g * L + iot], ones)
                            hrec[pl.ds(g * L, L)] = h

                    pltpu.async_copy(
                        buf,
                        out_hbm.at[pl.ds(f * NUM_BINS, NUM_BINS),
                                   pl.ds(b0, CB)],
                        souts[s])
                    nxt = item + 2 * N_OH_WORKERS

                    @pl.when(nxt < N_ITEMS)
                    def _prefetch():
                        _issue_in(nxt, s)

            return _

        lax.fori_loop(0, KMAX // 2, _pair, None)

        for s in range(2):
            item = wid + N_OH_WORKERS * (KMAX - 2 + s)

            @pl.when(item < N_ITEMS)
            def _drain():
                pltpu.make_async_copy(
                    bufs[s], out_hbm.at[pl.ds(0, NUM_BINS), pl.ds(0, CB)],
                    souts[s]).wait()


def kernel(int_features, float_features):
    xi = int_features.T.reshape(N_CAT * B)
    xf = float_features.T
    run = pl.kernel(
        _sc_body,
        out_type=jax.ShapeDtypeStruct((OUT_W, B), jnp.float32),
        mesh=plsc.VectorSubcoreMesh(core_axis_name="c", subcore_axis_name="s"),
        compiler_params=pltpu.CompilerParams(needs_layout_passes=False),
        scratch_types=[
            pltpu.VMEM((2 * CB,), jnp.int32),
            pltpu.VMEM((2 * CB,), jnp.int32),
            pltpu.VMEM((NUM_BINS, CB), jnp.float32),
            pltpu.VMEM((NUM_BINS, CB), jnp.float32),
            pltpu.VMEM((CB,), jnp.int32),
            pltpu.VMEM((CB,), jnp.int32),
            pltpu.VMEM((1, B), jnp.float32),
            pltpu.SemaphoreType.DMA,
            pltpu.SemaphoreType.DMA,
            pltpu.SemaphoreType.DMA,
            pltpu.SemaphoreType.DMA,
        ],
    )
    return run(xi, xf).T


# speedup vs baseline: 6.3003x; 1.0151x over previous
import jax
import jax.numpy as jnp
from jax import lax
from jax.experimental import pallas as pl
from jax.experimental.pallas import tpu as pltpu
from jax.experimental.pallas import tpu_sc as plsc

B = 16384
N_CAT = 26
N_DENSE = 13
NUM_BINS = 128
N_ONEHOT = N_CAT + 3
OUT_W = N_ONEHOT * NUM_BINS + N_DENSE

NC, NS, L = 2, 16, 16
NW = NC * NS
CB = 256
NCHUNK = B // CB
N_ITEMS = N_ONEHOT * NCHUNK
KMAX = N_ITEMS // NW
DENSE_ROW0 = N_ONEHOT * NUM_BINS


def _item_srcs(item):
    f = item >> 6
    b0 = (item & 63) * CB
    is_cross = f >= N_CAT
    p = f - N_CAT
    fa = jnp.where(is_cross, 2 * p, f)
    fb = jnp.where(is_cross, 2 * p + 1, f)
    return f, b0, fa, fb, is_cross


def _sc_body(xi_hbm, xf_hbm, out_hbm, stg0, stg1, buf0, buf1, hrec0, hrec1,
             dbuf, sin0, sin1, sout0, sout1):
    wid = lax.axis_index("s") * NC + lax.axis_index("c")
    iot = lax.iota(jnp.int32, L)
    ones = jnp.ones((L,), jnp.float32)
    zeros = jnp.zeros((L,), jnp.float32)
    izeros = jnp.zeros((L,), jnp.int32)

    stgs = (stg0, stg1)
    bufs = (buf0, buf1)
    hrecs = (hrec0, hrec1)
    sins = (sin0, sin1)
    souts = (sout0, sout1)

    def _zero(i, _):
        for u in range(8):
            g = i * 8 + u
            buf0[g >> 4, pl.ds((g & 15) * L, L)] = zeros
            buf1[g >> 4, pl.ds((g & 15) * L, L)] = zeros
        return _

    def _issue_in(item, s):
        _, b0, fa, fb, _ = _item_srcs(item)
        pltpu.async_copy(xi_hbm.at[pl.ds(fa, 1), pl.ds(b0, CB)],
                         stgs[s].at[pl.ds(0, 1), pl.ds(0, CB)], sins[s])
        pltpu.async_copy(xi_hbm.at[pl.ds(fb, 1), pl.ds(b0, CB)],
                         stgs[s].at[pl.ds(1, 1), pl.ds(0, CB)], sins[s])

    for s in range(2):
        _issue_in(wid + NW * s, s)

    lax.fori_loop(0, NUM_BINS * CB // L // 8, _zero, None)
    for g in range(CB // L):
        hrec0[pl.ds(g * L, L)] = izeros
        hrec1[pl.ds(g * L, L)] = izeros

    def _pair(kk, carry):
        for s in range(2):
            k = 2 * kk + s
            item = wid + NW * k
            buf, stg, hrec = bufs[s], stgs[s], hrecs[s]
            f, b0, _fa, _fb, is_cross = _item_srcs(item)
            pltpu.make_async_copy(
                xi_hbm.at[pl.ds(0, 2), pl.ds(0, CB)], stg, sins[s]).wait()

            @pl.when(k >= 2)
            def _cleanup():
                pltpu.make_async_copy(
                    buf, out_hbm.at[pl.ds(0, NUM_BINS), pl.ds(0, CB)],
                    souts[s]).wait()
                for g in range(CB // L):
                    hold = hrec[pl.ds(g * L, L)]
                    plsc.store_scatter(buf, [hold, g * L + iot], zeros)

            @pl.when(is_cross)
            def _cross():
                for g in range(CB // L):
                    xa = stg[0, pl.ds(g * L, L)]
                    xb = stg[1, pl.ds(g * L, L)]
                    comb = (xa % 32749) * 32749 + (xb % 32749)
                    h = (comb * 31 + 17) & 127
                    plsc.store_scatter(buf, [h, g * L + iot], ones)
                    hrec[pl.ds(g * L, L)] = h

            @pl.when(jnp.logical_not(is_cross))
            def _plain():
                for g in range(CB // L):
                    xa = stg[0, pl.ds(g * L, L)]
                    h = (xa * 31 + 17) & 127
                    plsc.store_scatter(buf, [h, g * L + iot], ones)
                    hrec[pl.ds(g * L, L)] = h

            pltpu.async_copy(
                buf,
                out_hbm.at[pl.ds(f * NUM_BINS, NUM_BINS), pl.ds(b0, CB)],
                souts[s])
            nxt = item + 2 * NW

            @pl.when(k < KMAX - 2)
            def _prefetch():
                _issue_in(nxt, s)

        return carry

    lax.fori_loop(0, KMAX // 2, _pair, None)

    @pl.when(wid < N_DENSE)
    def _dense():
        pltpu.sync_copy(xf_hbm.at[pl.ds(wid, 1), pl.ds(0, B)], dbuf)
        pltpu.sync_copy(dbuf, out_hbm.at[pl.ds(DENSE_ROW0 + wid, 1),
                                         pl.ds(0, B)])

    for s in range(2):
        pltpu.make_async_copy(
            bufs[s], out_hbm.at[pl.ds(0, NUM_BINS), pl.ds(0, CB)],
            souts[s]).wait()


def kernel(int_features, float_features):
    run = pl.kernel(
        _sc_body,
        out_type=jax.ShapeDtypeStruct((OUT_W, B), jnp.float32),
        mesh=plsc.VectorSubcoreMesh(core_axis_name="c", subcore_axis_name="s"),
        compiler_params=pltpu.CompilerParams(needs_layout_passes=False),
        scratch_types=[
            pltpu.VMEM((2, CB), jnp.int32),
            pltpu.VMEM((2, CB), jnp.int32),
            pltpu.VMEM((NUM_BINS, CB), jnp.float32),
            pltpu.VMEM((NUM_BINS, CB), jnp.float32),
            pltpu.VMEM((CB,), jnp.int32),
            pltpu.VMEM((CB,), jnp.int32),
            pltpu.VMEM((1, B), jnp.float32),
            pltpu.SemaphoreType.DMA,
            pltpu.SemaphoreType.DMA,
            pltpu.SemaphoreType.DMA,
            pltpu.SemaphoreType.DMA,
        ],
    )
    return run(int_features.T, float_features.T).T


# speedup vs baseline: 7.4420x; 1.1812x over previous
import jax
import jax.numpy as jnp
from jax import lax
from jax.experimental import pallas as pl
from jax.experimental.pallas import tpu as pltpu
from jax.experimental.pallas import tpu_sc as plsc

B = 16384
N_CAT = 26
N_DENSE = 13
NUM_BINS = 128
N_ONEHOT = N_CAT + 3
OUT_W = N_ONEHOT * NUM_BINS + N_DENSE

NC, NS, L = 2, 16, 16
NW = NC * NS
CB = 256
NCHUNK = B // CB
N_ITEMS = N_ONEHOT * NCHUNK
KMAX = N_ITEMS // NW
NBUF = 3
DENSE_ROW0 = N_ONEHOT * NUM_BINS


def _item_srcs(item):
    f = item >> 6
    b0 = (item & 63) * CB
    is_cross = f >= N_CAT
    p = f - N_CAT
    fa = jnp.where(is_cross, 2 * p, f)
    fb = jnp.where(is_cross, 2 * p + 1, f)
    return f, b0, fa, fb, is_cross


def _sc_body(xi_hbm, xf_hbm, out_hbm, stg0, stg1, stg2, buf0, buf1, buf2,
             hrec0, hrec1, hrec2, dbuf, sin0, sin1, sin2, sout0, sout1,
             sout2):
    wid = lax.axis_index("s") * NC + lax.axis_index("c")
    iot = lax.iota(jnp.int32, L)
    ones = jnp.ones((L,), jnp.float32)
    zeros = jnp.zeros((L,), jnp.float32)
    izeros = jnp.zeros((L,), jnp.int32)

    stgs = (stg0, stg1, stg2)
    bufs = (buf0, buf1, buf2)
    hrecs = (hrec0, hrec1, hrec2)
    sins = (sin0, sin1, sin2)
    souts = (sout0, sout1, sout2)

    def _issue_in(item, s):
        _, b0, fa, fb, _ = _item_srcs(item)
        pltpu.async_copy(xi_hbm.at[pl.ds(fa, 1), pl.ds(b0, CB)],
                         stgs[s].at[pl.ds(0, 1), pl.ds(0, CB)], sins[s])
        pltpu.async_copy(xi_hbm.at[pl.ds(fb, 1), pl.ds(b0, CB)],
                         stgs[s].at[pl.ds(1, 1), pl.ds(0, CB)], sins[s])

    for s in range(NBUF):
        _issue_in(wid + NW * s, s)

    def _zero(i, _):
        for u in range(8):
            g = i * 8 + u
            for buf in bufs:
                buf[g >> 4, pl.ds((g & 15) * L, L)] = zeros
        return _

    lax.fori_loop(0, NUM_BINS * CB // L // 8, _zero, None)
    for g in range(CB // L):
        for hrec in hrecs:
            hrec[pl.ds(g * L, L)] = izeros

    def _run_item(k, s):
        item = wid + NW * k
        buf, stg, hrec = bufs[s], stgs[s], hrecs[s]
        f, b0, _fa, _fb, is_cross = _item_srcs(item)
        pltpu.make_async_copy(
            xi_hbm.at[pl.ds(0, 2), pl.ds(0, CB)], stg, sins[s]).wait()

        @pl.when(k >= NBUF)
        def _cleanup():
            pltpu.make_async_copy(
                buf, out_hbm.at[pl.ds(0, NUM_BINS), pl.ds(0, CB)],
                souts[s]).wait()

            def _cl(g, c):
                hold = hrec[pl.ds(g * L, L)]
                plsc.store_scatter(buf, [hold, g * L + iot], zeros)
                return c

            lax.fori_loop(0, CB // L, _cl, None)

        @pl.when(is_cross)
        def _cross():
            def _cr(g, c):
                xa = stg[0, pl.ds(g * L, L)]
                xb = stg[1, pl.ds(g * L, L)]
                comb = (xa % 32749) * 32749 + (xb % 32749)
                h = (comb * 31 + 17) & 127
                plsc.store_scatter(buf, [h, g * L + iot], ones)
                hrec[pl.ds(g * L, L)] = h
                return c

            lax.fori_loop(0, CB // L, _cr, None)

        @pl.when(jnp.logical_not(is_cross))
        def _plain():
            def _pl(g, c):
                xa = stg[0, pl.ds(g * L, L)]
                h = (xa * 31 + 17) & 127
                plsc.store_scatter(buf, [h, g * L + iot], ones)
                hrec[pl.ds(g * L, L)] = h
                return c

            lax.fori_loop(0, CB // L, _pl, None)

        pltpu.async_copy(
            buf, out_hbm.at[pl.ds(f * NUM_BINS, NUM_BINS), pl.ds(b0, CB)],
            souts[s])

        @pl.when(k < KMAX - NBUF)
        def _prefetch():
            _issue_in(item + NBUF * NW, s)

    def _round(kk, carry):
        for s in range(NBUF):
            _run_item(kk * NBUF + s, s)
        return carry

    lax.fori_loop(0, KMAX // NBUF, _round, None)
    for t in range(KMAX - (KMAX // NBUF) * NBUF):
        _run_item(jnp.int32((KMAX // NBUF) * NBUF + t), t)

    @pl.when(wid < N_DENSE)
    def _dense():
        pltpu.sync_copy(xf_hbm.at[pl.ds(wid, 1), pl.ds(0, B)], dbuf)
        pltpu.sync_copy(dbuf, out_hbm.at[pl.ds(DENSE_ROW0 + wid, 1),
                                         pl.ds(0, B)])

    for s in range(NBUF):
        pltpu.make_async_copy(
            bufs[s], out_hbm.at[pl.ds(0, NUM_BINS), pl.ds(0, CB)],
            souts[s]).wait()


def kernel(int_features, float_features):
    run = pl.kernel(
        _sc_body,
        out_type=jax.ShapeDtypeStruct((OUT_W, B), jnp.float32),
        mesh=plsc.VectorSubcoreMesh(core_axis_name="c", subcore_axis_name="s"),
        compiler_params=pltpu.CompilerParams(needs_layout_passes=False),
        scratch_types=[
            pltpu.VMEM((2, CB), jnp.int32),
            pltpu.VMEM((2, CB), jnp.int32),
            pltpu.VMEM((2, CB), jnp.int32),
            pltpu.VMEM((NUM_BINS, CB), jnp.float32),
            pltpu.VMEM((NUM_BINS, CB), jnp.float32),
            pltpu.VMEM((NUM_BINS, CB), jnp.float32),
            pltpu.VMEM((CB,), jnp.int32),
            pltpu.VMEM((CB,), jnp.int32),
            pltpu.VMEM((CB,), jnp.int32),
            pltpu.VMEM((1, B), jnp.float32),
            pltpu.SemaphoreType.DMA,
            pltpu.SemaphoreType.DMA,
            pltpu.SemaphoreType.DMA,
            pltpu.SemaphoreType.DMA,
            pltpu.SemaphoreType.DMA,
            pltpu.SemaphoreType.DMA,
        ],
    )
    return run(int_features.T, float_features.T).T
